# Initial kernel scaffold; baseline (speedup 1.0000x reference)
import jax
import jax.numpy as jnp
from jax.experimental import pallas as pl


def kernel(x, edge_index, W1, b1, W2, b2):
    raise NotImplementedError("write your pallas kernel here")


# speedup vs baseline: 8.6949x; 8.6949x over previous
import functools
import jax
import jax.numpy as jnp
from jax import lax
from jax.experimental import pallas as pl
from jax.experimental.pallas import tpu as pltpu
from jax.experimental.pallas import tpu_sc as plsc

SUBCORES = 16
LANES = 16
CHUNK = 128


def _make_deg_kernel(n_pad, chunks_per_tile):
  per_tile = n_pad // SUBCORES
  mesh = plsc.VectorSubcoreMesh(core_axis_name="c", subcore_axis_name="s")

  @functools.partial(
      pl.kernel,
      out_type=jax.ShapeDtypeStruct((n_pad,), jnp.float32),
      mesh=mesh,
      scratch_types=[
          pltpu.VMEM_SHARED((n_pad,), jnp.float32),
          pltpu.VMEM((chunks_per_tile, CHUNK), jnp.int32),
          pltpu.VMEM((CHUNK,), jnp.float32),
          pltpu.VMEM((per_tile,), jnp.float32),
      ],
  )
  def deg_kernel(dst_hbm, deg_hbm, deg_sh, dst_v, ones_v, work_v):
    cid = lax.axis_index("c")
    tid = lax.axis_index("s")

    @pl.when(cid == 0)
    def _():
      base = tid * per_tile
      for i in range(per_tile // LANES):
        work_v[pl.ds(i * LANES, LANES)] = jnp.full((LANES,), 1.0, jnp.float32)
      pltpu.sync_copy(work_v, deg_sh.at[pl.ds(base, per_tile)])
      for i in range(CHUNK // LANES):
        ones_v[pl.ds(i * LANES, LANES)] = jnp.full((LANES,), 1.0, jnp.float32)
      pltpu.sync_copy(dst_hbm.at[tid], dst_v)
      plsc.subcore_barrier()

      @pl.loop(0, chunks_per_tile)
      def _(c):
        pltpu.sync_copy(ones_v, deg_sh.at[dst_v.at[c]], add=True)

      plsc.subcore_barrier()
      pltpu.sync_copy(deg_sh.at[pl.ds(base, per_tile)],
                      deg_hbm.at[pl.ds(base, per_tile)])

  return deg_kernel


def _make_agg_kernel(n_pad, d_half, chunks_per_tile):
  rows_per_tile = n_pad // SUBCORES
  hc = chunks_per_tile // 2
  mesh = plsc.VectorSubcoreMesh(core_axis_name="c", subcore_axis_name="s")

  @functools.partial(
      pl.kernel,
      out_type=jax.ShapeDtypeStruct((2, n_pad, d_half), jnp.float32),
      mesh=mesh,
      scratch_types=[
          pltpu.VMEM_SHARED((n_pad, d_half), jnp.float32),
          pltpu.VMEM((hc, CHUNK), jnp.int32),
          pltpu.VMEM((hc, CHUNK), jnp.int32),
          pltpu.VMEM((CHUNK, d_half), jnp.float32),
          pltpu.VMEM((CHUNK, d_half), jnp.float32),
          pltpu.SemaphoreType.DMA,
          pltpu.SemaphoreType.DMA,
      ],
  )
  def agg_kernel(src_hbm, dst_hbm, hs_lo, hs_hi, out_hbm,
                 acc_sh, src_v, dst_v, buf_a, buf_b, sem_a, sem_b):
    cid = lax.axis_index("c")
    tid = lax.axis_index("s")
    base = tid * rows_per_tile
    rows = pl.ds(base, rows_per_tile)

    def run(hs_ref):
      pltpu.sync_copy(hs_ref.at[rows], acc_sh.at[rows])
      plsc.subcore_barrier()

      for ph in range(2):
        pltpu.sync_copy(src_hbm.at[tid, pl.ds(ph * hc, hc)], src_v)
        pltpu.sync_copy(dst_hbm.at[tid, pl.ds(ph * hc, hc)], dst_v)

        pltpu.async_copy(hs_ref.at[src_v.at[0]], buf_a, sem_a)

        @pl.loop(0, hc, step=2)
        def _(c):
          pltpu.make_async_copy(hs_ref.at[src_v.at[c]], buf_a, sem_a).wait()

          @pl.when(c + 1 < hc)
          def _():
            pltpu.async_copy(hs_ref.at[src_v.at[c + 1]], buf_b, sem_b)

          pltpu.sync_copy(buf_a, acc_sh.at[dst_v.at[c]], add=True)

          @pl.when(c + 1 < hc)
          def _():
            pltpu.make_async_copy(
                hs_ref.at[src_v.at[c + 1]], buf_b, sem_b).wait()

            @pl.when(c + 2 < hc)
            def _():
              pltpu.async_copy(hs_ref.at[src_v.at[c + 2]], buf_a, sem_a)

            pltpu.sync_copy(buf_b, acc_sh.at[dst_v.at[c + 1]], add=True)

      plsc.subcore_barrier()
      pltpu.sync_copy(acc_sh.at[rows], out_hbm.at[cid].at[rows])

    @pl.when(cid == 0)
    def _():
      run(hs_lo)

    @pl.when(cid == 1)
    def _():
      run(hs_hi)

  return agg_kernel


def _mm1_body(x_ref, w_ref, deg_ref, lo_ref, hi_ref):
  h = jnp.dot(x_ref[...], w_ref[...], preferred_element_type=jnp.float32)
  hs = h * lax.rsqrt(deg_ref[...])
  d = h.shape[1] // 2
  lo_ref[...] = hs[:, :d]
  hi_ref[...] = hs[:, d:]


def _mm2_body(acc_ref, deg_ref, b_ref, w_ref, lo_ref, hi_ref):
  dinv = lax.rsqrt(deg_ref[...])
  t = jnp.concatenate([acc_ref[0], acc_ref[1]], axis=1)
  p = jnp.maximum(t * dinv + b_ref[...], 0.0)
  h = jnp.dot(p, w_ref[...], preferred_element_type=jnp.float32)
  hs = h * dinv
  d = h.shape[1] // 2
  lo_ref[...] = hs[:, :d]
  hi_ref[...] = hs[:, d:]


def _post_body(acc_ref, deg_ref, b_ref, out_ref):
  t = jnp.concatenate([acc_ref[0], acc_ref[1]], axis=1)
  out_ref[...] = jnp.maximum(t * lax.rsqrt(deg_ref[...]) + b_ref[...], 0.0)


def kernel(x, edge_index, W1, b1, W2, b2):
  n, d_in = x.shape
  d_h = W1.shape[1]
  d_half = d_h // 2
  e = edge_index.shape[1]

  n_pad = -(-n // (SUBCORES * LANES)) * SUBCORES * LANES
  x_pad = jnp.pad(x, ((0, n_pad - n), (0, 0)))

  chunks_per_tile = -(-e // (SUBCORES * CHUNK * 16)) * 16
  e_pad = SUBCORES * chunks_per_tile * CHUNK
  src = edge_index[0]
  dst = edge_index[1]
  if e_pad != e:
    pad = e_pad - e
    src = jnp.concatenate([src, jnp.zeros((pad,), jnp.int32)])
    dst = jnp.concatenate([dst, jnp.full((pad,), n, jnp.int32)])
  src_r = src.reshape(SUBCORES, chunks_per_tile, CHUNK)
  dst_r = dst.reshape(SUBCORES, chunks_per_tile, CHUNK)

  deg_full = _make_deg_kernel(n_pad, chunks_per_tile)(dst_r)
  deg2d = deg_full.reshape(n_pad, 1)

  blk = 1024
  grid = n_pad // blk
  mm1 = pl.pallas_call(
      _mm1_body,
      grid=(grid,),
      in_specs=[
          pl.BlockSpec((blk, d_in), lambda i: (i, 0)),
          pl.BlockSpec((d_in, d_h), lambda i: (0, 0)),
          pl.BlockSpec((blk, 1), lambda i: (i, 0)),
      ],
      out_specs=[
          pl.BlockSpec((blk, d_half), lambda i: (i, 0)),
          pl.BlockSpec((blk, d_half), lambda i: (i, 0)),
      ],
      out_shape=[
          jax.ShapeDtypeStruct((n_pad, d_half), jnp.float32),
          jax.ShapeDtypeStruct((n_pad, d_half), jnp.float32),
      ],
  )
  mm2 = pl.pallas_call(
      _mm2_body,
      grid=(grid,),
      in_specs=[
          pl.BlockSpec((2, blk, d_half), lambda i: (0, i, 0)),
          pl.BlockSpec((blk, 1), lambda i: (i, 0)),
          pl.BlockSpec((1, d_h), lambda i: (0, 0)),
          pl.BlockSpec((d_h, d_h), lambda i: (0, 0)),
      ],
      out_specs=[
          pl.BlockSpec((blk, d_half), lambda i: (i, 0)),
          pl.BlockSpec((blk, d_half), lambda i: (i, 0)),
      ],
      out_shape=[
          jax.ShapeDtypeStruct((n_pad, d_half), jnp.float32),
          jax.ShapeDtypeStruct((n_pad, d_half), jnp.float32),
      ],
  )
  post = pl.pallas_call(
      _post_body,
      grid=(grid,),
      in_specs=[
          pl.BlockSpec((2, blk, d_half), lambda i: (0, i, 0)),
          pl.BlockSpec((blk, 1), lambda i: (i, 0)),
          pl.BlockSpec((1, d_h), lambda i: (0, 0)),
      ],
      out_specs=pl.BlockSpec((blk, d_h), lambda i: (i, 0)),
      out_shape=jax.ShapeDtypeStruct((n_pad, d_h), jnp.float32),
  )

  agg = _make_agg_kernel(n_pad, d_half, chunks_per_tile)

  hs1_lo, hs1_hi = mm1(x_pad, W1, deg2d)
  acc1 = agg(src_r, dst_r, hs1_lo, hs1_hi)
  hs2_lo, hs2_hi = mm2(acc1, deg2d, b1.reshape(1, d_h), W2)
  acc2 = agg(src_r, dst_r, hs2_lo, hs2_hi)
  return post(acc2, deg2d, b2.reshape(1, d_h))[:n]


# speedup vs baseline: 8.7687x; 1.0085x over previous
import functools
import jax
import jax.numpy as jnp
from jax import lax
from jax.experimental import pallas as pl
from jax.experimental.pallas import tpu as pltpu
from jax.experimental.pallas import tpu_sc as plsc

SUBCORES = 16
LANES = 16
DEG_CHUNK = 128
CHUNK = 64
NSTREAM = 4


def _make_deg_kernel(n_pad, chunks_per_tile):
  per_tile = n_pad // SUBCORES
  mesh = plsc.VectorSubcoreMesh(core_axis_name="c", subcore_axis_name="s")

  @functools.partial(
      pl.kernel,
      out_type=jax.ShapeDtypeStruct((n_pad,), jnp.float32),
      mesh=mesh,
      scratch_types=[
          pltpu.VMEM_SHARED((n_pad,), jnp.float32),
          pltpu.VMEM((chunks_per_tile, DEG_CHUNK), jnp.int32),
          pltpu.VMEM((DEG_CHUNK,), jnp.float32),
          pltpu.VMEM((per_tile,), jnp.float32),
      ],
  )
  def deg_kernel(dst_hbm, deg_hbm, deg_sh, dst_v, ones_v, work_v):
    cid = lax.axis_index("c")
    tid = lax.axis_index("s")

    @pl.when(cid == 0)
    def _():
      base = tid * per_tile
      for i in range(per_tile // LANES):
        work_v[pl.ds(i * LANES, LANES)] = jnp.full((LANES,), 1.0, jnp.float32)
      pltpu.sync_copy(work_v, deg_sh.at[pl.ds(base, per_tile)])
      for i in range(DEG_CHUNK // LANES):
        ones_v[pl.ds(i * LANES, LANES)] = jnp.full((LANES,), 1.0, jnp.float32)
      pltpu.sync_copy(dst_hbm.at[tid], dst_v)
      plsc.subcore_barrier()

      @pl.loop(0, chunks_per_tile)
      def _(c):
        pltpu.sync_copy(ones_v, deg_sh.at[dst_v.at[c]], add=True)

      plsc.subcore_barrier()
      pltpu.sync_copy(deg_sh.at[pl.ds(base, per_tile)],
                      deg_hbm.at[pl.ds(base, per_tile)])

  return deg_kernel


def _make_agg_kernel(n_pad, d_half, chunks_per_tile):
  rows_per_tile = n_pad // SUBCORES
  phases = 4
  hc = chunks_per_tile // phases
  mesh = plsc.VectorSubcoreMesh(core_axis_name="c", subcore_axis_name="s")

  @functools.partial(
      pl.kernel,
      out_type=jax.ShapeDtypeStruct((2, n_pad, d_half), jnp.float32),
      mesh=mesh,
      scratch_types=[
          pltpu.VMEM_SHARED((n_pad, d_half), jnp.float32),
          pltpu.VMEM((hc, CHUNK), jnp.int32),
          pltpu.VMEM((hc, CHUNK), jnp.int32),
          [pltpu.VMEM((CHUNK, d_half), jnp.float32)] * NSTREAM,
          [pltpu.SemaphoreType.DMA] * NSTREAM,
      ],
  )
  def agg_kernel(src_hbm, dst_hbm, hs_lo, hs_hi, out_hbm,
                 acc_sh, src_v, dst_v, bufs, sems):
    cid = lax.axis_index("c")
    tid = lax.axis_index("s")
    base = tid * rows_per_tile
    rows = pl.ds(base, rows_per_tile)

    def run(hs_ref):
      pltpu.sync_copy(hs_ref.at[rows], acc_sh.at[rows])
      plsc.subcore_barrier()

      for ph in range(phases):
        pltpu.sync_copy(src_hbm.at[tid, pl.ds(ph * hc, hc)], src_v)
        pltpu.sync_copy(dst_hbm.at[tid, pl.ds(ph * hc, hc)], dst_v)

        for b in range(NSTREAM):
          pltpu.async_copy(hs_ref.at[src_v.at[b]], bufs[b], sems[b])

        @pl.loop(0, hc, step=NSTREAM)
        def _(c):
          for b in range(NSTREAM):
            pltpu.make_async_copy(
                hs_ref.at[src_v.at[c + b]], bufs[b], sems[b]).wait()
            pltpu.sync_copy(bufs[b], acc_sh.at[dst_v.at[c + b]], add=True)

            @pl.when(c + b + NSTREAM < hc)
            def _():
              pltpu.async_copy(
                  hs_ref.at[src_v.at[c + b + NSTREAM]], bufs[b], sems[b])

      plsc.subcore_barrier()
      pltpu.sync_copy(acc_sh.at[rows], out_hbm.at[cid].at[rows])

    @pl.when(cid == 0)
    def _():
      run(hs_lo)

    @pl.when(cid == 1)
    def _():
      run(hs_hi)

  return agg_kernel


def _mm1_body(x_ref, w_ref, deg_ref, lo_ref, hi_ref):
  h = jnp.dot(x_ref[...], w_ref[...], preferred_element_type=jnp.float32)
  hs = h * lax.rsqrt(deg_ref[...])
  d = h.shape[1] // 2
  lo_ref[...] = hs[:, :d]
  hi_ref[...] = hs[:, d:]


def _mm2_body(acc_ref, deg_ref, b_ref, w_ref, lo_ref, hi_ref):
  dinv = lax.rsqrt(deg_ref[...])
  t = jnp.concatenate([acc_ref[0], acc_ref[1]], axis=1)
  p = jnp.maximum(t * dinv + b_ref[...], 0.0)
  h = jnp.dot(p, w_ref[...], preferred_element_type=jnp.float32)
  hs = h * dinv
  d = h.shape[1] // 2
  lo_ref[...] = hs[:, :d]
  hi_ref[...] = hs[:, d:]


def _post_body(acc_ref, deg_ref, b_ref, out_ref):
  t = jnp.concatenate([acc_ref[0], acc_ref[1]], axis=1)
  out_ref[...] = jnp.maximum(t * lax.rsqrt(deg_ref[...]) + b_ref[...], 0.0)


def kernel(x, edge_index, W1, b1, W2, b2):
  n, d_in = x.shape
  d_h = W1.shape[1]
  d_half = d_h // 2
  e = edge_index.shape[1]

  n_pad = -(-n // (SUBCORES * LANES)) * SUBCORES * LANES
  x_pad = jnp.pad(x, ((0, n_pad - n), (0, 0)))

  chunks_per_tile = -(-e // (SUBCORES * CHUNK * 16)) * 16
  e_pad = SUBCORES * chunks_per_tile * CHUNK
  src = edge_index[0]
  dst = edge_index[1]
  if e_pad != e:
    pad = e_pad - e
    src = jnp.concatenate([src, jnp.zeros((pad,), jnp.int32)])
    dst = jnp.concatenate([dst, jnp.full((pad,), n, jnp.int32)])
  src_r = src.reshape(SUBCORES, chunks_per_tile, CHUNK)
  dst_r = dst.reshape(SUBCORES, chunks_per_tile, CHUNK)
  deg_chunks = e_pad // (SUBCORES * DEG_CHUNK)
  dst_deg = dst.reshape(SUBCORES, deg_chunks, DEG_CHUNK)

  deg_full = _make_deg_kernel(n_pad, deg_chunks)(dst_deg)
  deg2d = deg_full.reshape(n_pad, 1)

  blk = 1024
  grid = n_pad // blk
  mm1 = pl.pallas_call(
      _mm1_body,
      grid=(grid,),
      in_specs=[
          pl.BlockSpec((blk, d_in), lambda i: (i, 0)),
          pl.BlockSpec((d_in, d_h), lambda i: (0, 0)),
          pl.BlockSpec((blk, 1), lambda i: (i, 0)),
      ],
      out_specs=[
          pl.BlockSpec((blk, d_half), lambda i: (i, 0)),
          pl.BlockSpec((blk, d_half), lambda i: (i, 0)),
      ],
      out_shape=[
          jax.ShapeDtypeStruct((n_pad, d_half), jnp.float32),
          jax.ShapeDtypeStruct((n_pad, d_half), jnp.float32),
      ],
  )
  mm2 = pl.pallas_call(
      _mm2_body,
      grid=(grid,),
      in_specs=[
          pl.BlockSpec((2, blk, d_half), lambda i: (0, i, 0)),
          pl.BlockSpec((blk, 1), lambda i: (i, 0)),
          pl.BlockSpec((1, d_h), lambda i: (0, 0)),
          pl.BlockSpec((d_h, d_h), lambda i: (0, 0)),
      ],
      out_specs=[
          pl.BlockSpec((blk, d_half), lambda i: (i, 0)),
          pl.BlockSpec((blk, d_half), lambda i: (i, 0)),
      ],
      out_shape=[
          jax.ShapeDtypeStruct((n_pad, d_half), jnp.float32),
          jax.ShapeDtypeStruct((n_pad, d_half), jnp.float32),
      ],
  )
  post = pl.pallas_call(
      _post_body,
      grid=(grid,),
      in_specs=[
          pl.BlockSpec((2, blk, d_half), lambda i: (0, i, 0)),
          pl.BlockSpec((blk, 1), lambda i: (i, 0)),
          pl.BlockSpec((1, d_h), lambda i: (0, 0)),
      ],
      out_specs=pl.BlockSpec((blk, d_h), lambda i: (i, 0)),
      out_shape=jax.ShapeDtypeStruct((n_pad, d_h), jnp.float32),
  )

  agg = _make_agg_kernel(n_pad, d_half, chunks_per_tile)

  hs1_lo, hs1_hi = mm1(x_pad, W1, deg2d)
  acc1 = agg(src_r, dst_r, hs1_lo, hs1_hi)
  hs2_lo, hs2_hi = mm2(acc1, deg2d, b1.reshape(1, d_h), W2)
  acc2 = agg(src_r, dst_r, hs2_lo, hs2_hi)
  return post(acc2, deg2d, b2.reshape(1, d_h))[:n]
